# baseline (speedup 1.0000x reference)
import functools

import jax
import jax.numpy as jnp
from jax import lax
from jax.experimental import pallas as pl
from jax.experimental.pallas import tpu as pltpu
from jax.experimental.pallas import tpu_sc as plsc

N = 10000
E = 320000
D = 128
H = 128

NC = 2
NS = 16
NW = NC * NS
L = 16

EPT = E // NW
CH = 80
NCHUNK = EPT // CH
WPT = 624
TAIL = N - NS * WPT
ZROWS = 16

_mesh = plsc.VectorSubcoreMesh(core_axis_name="c", subcore_axis_name="s")
_sc_params = pltpu.CompilerParams(needs_layout_passes=False)

_Z16 = functools.partial(jnp.zeros, (L,), jnp.float32)


def _zero_1d(ref, nwords):
    def body(i, c):
        ref[pl.ds(i * L, L)] = _Z16()
        return c
    lax.fori_loop(0, nwords // L, body, 0)


@functools.partial(
    pl.kernel,
    out_type=jax.ShapeDtypeStruct((NW, N), jnp.float32),
    mesh=_mesh,
    compiler_params=_sc_params,
    scratch_types=[
        pltpu.VMEM((NCHUNK, 2, CH), jnp.int32),
        pltpu.VMEM((N,), jnp.float32),
    ],
)
def _sc_deg(es_hbm, out_hbm, es_v, acc_v):
    c = lax.axis_index("c")
    s = lax.axis_index("s")
    wid = c * NS + s
    _zero_1d(acc_v, N)
    pltpu.sync_copy(es_hbm.at[wid], es_v)
    ones = jnp.ones((L,), jnp.float32)

    def body(k, carry):
        for j in range(CH // L):
            di = es_v[k, 1, pl.ds(j * L, L)]
            plsc.addupdate_scatter(acc_v, [di], ones)
        return carry

    lax.fori_loop(0, NCHUNK, body, 0)
    pltpu.sync_copy(acc_v, out_hbm.at[wid])


_RING = 4


@functools.partial(
    pl.kernel,
    out_type=jax.ShapeDtypeStruct((NC, N, D), jnp.float32),
    mesh=_mesh,
    compiler_params=_sc_params,
    scratch_types=(
        [pltpu.VMEM_SHARED((N, D), jnp.float32)]
        + [pltpu.VMEM((CH, D), jnp.float32)] * _RING
        + [pltpu.VMEM((2, CH), jnp.int32)] * _RING
        + [pltpu.VMEM((ZROWS, D), jnp.float32)]
        + [pltpu.SemaphoreType.DMA] * (2 * _RING)
    ),
)
def _sc_rows(p1_hbm, es_hbm, out_hbm, acc,
             rows0, rows1, rows2, rows3, idx0, idx1, idx2, idx3, zbuf,
             g0, g1, g2, g3, i0, i1, i2, i3):
    c = lax.axis_index("c")
    s = lax.axis_index("s")
    wid = c * NS + s
    bufs = ((rows0, idx0, g0, i0), (rows1, idx1, g1, i1),
            (rows2, idx2, g2, i2), (rows3, idx3, g3, i3))

    def _iload(k, ib, isem):
        pltpu.async_copy(es_hbm.at[wid, k], ib, isem)

    def _iwait(k, ib, isem):
        pltpu.make_async_copy(es_hbm.at[wid, k], ib, isem).wait()

    def _gissue(ib, rb, gsem):
        pltpu.async_copy(p1_hbm.at[ib.at[0]], rb, gsem)

    def _gwait(ib, rb, gsem):
        pltpu.make_async_copy(p1_hbm.at[ib.at[0]], rb, gsem).wait()

    def _scat(ib, rb):
        pltpu.sync_copy(rb, acc.at[ib.at[1]], add=True)

    for b in range(_RING):
        _iload(b, bufs[b][1], bufs[b][3])
    for b in range(2):
        _iwait(b, bufs[b][1], bufs[b][3])
        _gissue(bufs[b][1], bufs[b][0], bufs[b][2])

    def zrow(i, carry):
        for j in range(D // L):
            zbuf[i, pl.ds(j * L, L)] = _Z16()
        return carry

    lax.fori_loop(0, ZROWS, zrow, 0)
    for r in range(WPT // ZROWS):
        pltpu.sync_copy(zbuf, acc.at[pl.ds(s * WPT + r * ZROWS, ZROWS)])

    @pl.when(s == NS - 1)
    def _zero_tail():
        pltpu.sync_copy(zbuf.at[pl.ds(0, TAIL)], acc.at[pl.ds(NS * WPT, TAIL)])

    plsc.subcore_barrier()


    def body(i, carry):
        for b in range(_RING):
            kk = _RING * i + b
            rb, ib, gs, isem = bufs[b]
            rb2, ib2, gs2, isem2 = bufs[(b + 2) % _RING]
            _gwait(ib, rb, gs)
            _scat(ib, rb)

            @pl.when(kk <= NCHUNK - 1 - _RING)
            def _prefetch_idx():
                _iload(kk + _RING, ib, isem)

            @pl.when(kk <= NCHUNK - 3)
            def _prefetch_rows():
                _iwait(kk + 2, ib2, isem2)
                _gissue(ib2, rb2, gs2)
        return carry

    lax.fori_loop(0, NCHUNK // _RING, body, 0)
    for kk in range(_RING * (NCHUNK // _RING), NCHUNK):
        rb, ib, gs, isem = bufs[kk % _RING]
        _gwait(ib, rb, gs)
        _scat(ib, rb)
    plsc.subcore_barrier()
    pltpu.sync_copy(acc.at[pl.ds(s * WPT, WPT)],
                    out_hbm.at[c, pl.ds(s * WPT, WPT)])

    @pl.when(s == NS - 1)
    def _write_tail():
        pltpu.sync_copy(acc.at[pl.ds(NS * WPT, TAIL)],
                        out_hbm.at[c, pl.ds(NS * WPT, TAIL)])


@functools.partial(
    pl.kernel,
    out_type=jax.ShapeDtypeStruct((NW, N), jnp.float32),
    mesh=_mesh,
    compiler_params=_sc_params,
    scratch_types=[
        pltpu.VMEM((NCHUNK, 2, CH), jnp.int32),
        pltpu.VMEM((N,), jnp.float32),
        pltpu.VMEM((N,), jnp.float32),
    ],
)
def _sc_scalar(es_hbm, p2_hbm, out_hbm, es_v, p2_v, acc_v):
    c = lax.axis_index("c")
    s = lax.axis_index("s")
    wid = c * NS + s
    _zero_1d(acc_v, N)
    pltpu.sync_copy(p2_hbm, p2_v)
    pltpu.sync_copy(es_hbm.at[wid], es_v)

    def body(k, carry):
        for j in range(CH // L):
            si = es_v[k, 0, pl.ds(j * L, L)]
            di = es_v[k, 1, pl.ds(j * L, L)]
            vals = plsc.load_gather(p2_v, [si])
            plsc.addupdate_scatter(acc_v, [di], vals)
        return carry

    lax.fori_loop(0, NCHUNK, body, 0)
    pltpu.sync_copy(acc_v, out_hbm.at[wid])


_RB = 2000


def _tc1_body(degp_ref, x_ref, w1_ref, p1_ref, dis_ref):
    deg = jnp.sum(degp_ref[...], axis=1) + 1.0
    dis = lax.rsqrt(deg)
    h = jnp.dot(x_ref[...], w1_ref[...], preferred_element_type=jnp.float32)
    p1_ref[...] = h * dis[:, None]
    dis_ref[...] = dis[:, None]


def _tc2_body(aggp_ref, p1_ref, dis_ref, b1_ref, w2_ref, p2_ref):
    a = aggp_ref[0] + aggp_ref[1] + p1_ref[...]
    o1 = a * dis_ref[...] + b1_ref[...]
    r = jnp.maximum(o1, 0.0)
    h2 = jnp.dot(r, w2_ref[...], preferred_element_type=jnp.float32)
    p2_ref[...] = h2 * dis_ref[...]


def _tc3_body(a2p_ref, p2_ref, dis_ref, b2_ref, out_ref):
    ssum = jnp.sum(a2p_ref[...], axis=1)[:, None] + p2_ref[...]
    out_ref[...] = ssum * dis_ref[...] + b2_ref[0, 0]


def kernel(x, edge_index, W1, b1, W2, b2):
    es = edge_index.reshape(2, NW, NCHUNK, CH).transpose(1, 2, 0, 3)

    deg_parts = _sc_deg(es)

    p1, dis = pl.pallas_call(
        _tc1_body,
        grid=(N // _RB,),
        in_specs=[
            pl.BlockSpec((_RB, NW), lambda i: (i, 0)),
            pl.BlockSpec((_RB, D), lambda i: (i, 0)),
            pl.BlockSpec((D, H), lambda i: (0, 0)),
        ],
        out_specs=[
            pl.BlockSpec((_RB, H), lambda i: (i, 0)),
            pl.BlockSpec((_RB, 1), lambda i: (i, 0)),
        ],
        out_shape=[
            jax.ShapeDtypeStruct((N, H), jnp.float32),
            jax.ShapeDtypeStruct((N, 1), jnp.float32),
        ],
    )(deg_parts.T, x, W1)

    agg_parts = _sc_rows(p1, es)

    p2 = pl.pallas_call(
        _tc2_body,
        grid=(N // _RB,),
        in_specs=[
            pl.BlockSpec((NC, _RB, H), lambda i: (0, i, 0)),
            pl.BlockSpec((_RB, H), lambda i: (i, 0)),
            pl.BlockSpec((_RB, 1), lambda i: (i, 0)),
            pl.BlockSpec((1, H), lambda i: (0, 0)),
            pl.BlockSpec((H, 1), lambda i: (0, 0)),
        ],
        out_specs=pl.BlockSpec((_RB, 1), lambda i: (i, 0)),
        out_shape=jax.ShapeDtypeStruct((N, 1), jnp.float32),
    )(agg_parts, p1, dis, b1.reshape(1, H), W2)

    agg2_parts = _sc_scalar(es, p2.reshape(N))

    out = pl.pallas_call(
        _tc3_body,
        grid=(N // _RB,),
        in_specs=[
            pl.BlockSpec((_RB, NW), lambda i: (i, 0)),
            pl.BlockSpec((_RB, 1), lambda i: (i, 0)),
            pl.BlockSpec((_RB, 1), lambda i: (i, 0)),
            pl.BlockSpec((1, 1), lambda i: (0, 0)),
        ],
        out_specs=pl.BlockSpec((_RB, 1), lambda i: (i, 0)),
        out_shape=jax.ShapeDtypeStruct((N, 1), jnp.float32),
    )(agg2_parts.T, p2, dis, b2.reshape(1, 1))

    return out.reshape(N)


# speedup vs baseline: 1.0028x; 1.0028x over previous
import functools

import jax
import jax.numpy as jnp
from jax import lax
from jax.experimental import pallas as pl
from jax.experimental.pallas import tpu as pltpu
from jax.experimental.pallas import tpu_sc as plsc

N = 10000
E = 320000
D = 128
H = 128

NC = 2
NS = 16
NW = NC * NS
L = 16

EPT = E // NW
CH = 80
NCHUNK = EPT // CH
WPT = 624
TAIL = N - NS * WPT
ZROWS = 16

_mesh = plsc.VectorSubcoreMesh(core_axis_name="c", subcore_axis_name="s")
_sc_params = pltpu.CompilerParams(needs_layout_passes=False)

_Z16 = functools.partial(jnp.zeros, (L,), jnp.float32)


def _zero_1d(ref, nwords):
    def body(i, c):
        ref[pl.ds(i * L, L)] = _Z16()
        return c
    lax.fori_loop(0, nwords // L, body, 0)


@functools.partial(
    pl.kernel,
    out_type=jax.ShapeDtypeStruct((NW, N), jnp.float32),
    mesh=_mesh,
    compiler_params=_sc_params,
    scratch_types=[
        pltpu.VMEM((NCHUNK, 2, CH), jnp.int32),
        pltpu.VMEM((N,), jnp.float32),
    ],
)
def _sc_deg(es_hbm, out_hbm, es_v, acc_v):
    c = lax.axis_index("c")
    s = lax.axis_index("s")
    wid = c * NS + s
    _zero_1d(acc_v, N)
    pltpu.sync_copy(es_hbm.at[wid], es_v)
    ones = jnp.ones((L,), jnp.float32)

    def body(k, carry):
        for j in range(CH // L):
            di = es_v[k, 1, pl.ds(j * L, L)]
            plsc.addupdate_scatter(acc_v, [di], ones)
        return carry

    lax.fori_loop(0, NCHUNK, body, 0)
    pltpu.sync_copy(acc_v, out_hbm.at[wid])


_RING = 3


@functools.partial(
    pl.kernel,
    out_type=jax.ShapeDtypeStruct((NC, N, D), jnp.float32),
    mesh=_mesh,
    compiler_params=_sc_params,
    scratch_types=(
        [pltpu.VMEM_SHARED((N, D), jnp.float32)]
        + [pltpu.VMEM((CH, D), jnp.float32)] * _RING
        + [pltpu.VMEM((2, CH), jnp.int32)] * _RING
        + [pltpu.VMEM((ZROWS, D), jnp.float32)]
        + [pltpu.SemaphoreType.DMA] * (2 * _RING)
    ),
)
def _sc_rows(p1_hbm, es_hbm, out_hbm, acc,
             rows0, rows1, rows2, idx0, idx1, idx2, zbuf,
             g0, g1, g2, i0, i1, i2):
    c = lax.axis_index("c")
    s = lax.axis_index("s")
    wid = c * NS + s
    bufs = ((rows0, idx0, g0, i0), (rows1, idx1, g1, i1), (rows2, idx2, g2, i2))

    def _iload(k, ib, isem):
        pltpu.async_copy(es_hbm.at[wid, k], ib, isem)

    def _iwait(k, ib, isem):
        pltpu.make_async_copy(es_hbm.at[wid, k], ib, isem).wait()

    def _gissue(ib, rb, gsem):
        pltpu.async_copy(p1_hbm.at[ib.at[0]], rb, gsem)

    def _gwait(ib, rb, gsem):
        pltpu.make_async_copy(p1_hbm.at[ib.at[0]], rb, gsem).wait()

    def _scat(ib, rb):
        pltpu.sync_copy(rb, acc.at[ib.at[1]], add=True)

    for b in range(_RING):
        _iload(b, bufs[b][1], bufs[b][3])
    for b in range(2):
        _iwait(b, bufs[b][1], bufs[b][3])
        _gissue(bufs[b][1], bufs[b][0], bufs[b][2])

    def zrow(i, carry):
        for j in range(D // L):
            zbuf[i, pl.ds(j * L, L)] = _Z16()
        return carry

    lax.fori_loop(0, ZROWS, zrow, 0)
    for r in range(WPT // ZROWS):
        pltpu.sync_copy(zbuf, acc.at[pl.ds(s * WPT + r * ZROWS, ZROWS)])

    @pl.when(s == NS - 1)
    def _zero_tail():
        pltpu.sync_copy(zbuf.at[pl.ds(0, TAIL)], acc.at[pl.ds(NS * WPT, TAIL)])

    plsc.subcore_barrier()


    def body(i, carry):
        for b in range(_RING):
            kk = _RING * i + b
            rb, ib, gs, isem = bufs[b]
            rb2, ib2, gs2, isem2 = bufs[(b + 2) % _RING]
            _gwait(ib, rb, gs)
            _scat(ib, rb)

            @pl.when(kk <= NCHUNK - 1 - _RING)
            def _prefetch_idx():
                _iload(kk + _RING, ib, isem)

            @pl.when(kk <= NCHUNK - 3)
            def _prefetch_rows():
                _iwait(kk + 2, ib2, isem2)
                _gissue(ib2, rb2, gs2)
        return carry

    lax.fori_loop(0, NCHUNK // _RING, body, 0)
    for kk in range(_RING * (NCHUNK // _RING), NCHUNK):
        rb, ib, gs, isem = bufs[kk % _RING]
        _gwait(ib, rb, gs)
        _scat(ib, rb)
    plsc.subcore_barrier()
    pltpu.sync_copy(acc.at[pl.ds(s * WPT, WPT)],
                    out_hbm.at[c, pl.ds(s * WPT, WPT)])

    @pl.when(s == NS - 1)
    def _write_tail():
        pltpu.sync_copy(acc.at[pl.ds(NS * WPT, TAIL)],
                        out_hbm.at[c, pl.ds(NS * WPT, TAIL)])


@functools.partial(
    pl.kernel,
    out_type=jax.ShapeDtypeStruct((NW, N), jnp.float32),
    mesh=_mesh,
    compiler_params=_sc_params,
    scratch_types=[
        pltpu.VMEM((NCHUNK, 2, CH), jnp.int32),
        pltpu.VMEM((N,), jnp.float32),
        pltpu.VMEM((N,), jnp.float32),
    ],
)
def _sc_scalar(es_hbm, p2_hbm, out_hbm, es_v, p2_v, acc_v):
    c = lax.axis_index("c")
    s = lax.axis_index("s")
    wid = c * NS + s
    _zero_1d(acc_v, N)
    pltpu.sync_copy(p2_hbm, p2_v)
    pltpu.sync_copy(es_hbm.at[wid], es_v)

    def body(k, carry):
        for j in range(CH // L):
            si = es_v[k, 0, pl.ds(j * L, L)]
            di = es_v[k, 1, pl.ds(j * L, L)]
            vals = plsc.load_gather(p2_v, [si])
            plsc.addupdate_scatter(acc_v, [di], vals)
        return carry

    lax.fori_loop(0, NCHUNK, body, 0)
    pltpu.sync_copy(acc_v, out_hbm.at[wid])


_RB = 2000


def _tc0_body(x_ref, w1_ref, h1_ref):
    h1_ref[...] = jnp.dot(x_ref[...], w1_ref[...],
                          preferred_element_type=jnp.float32)


def _tc1_body(degp_ref, h1_ref, p1_ref, dis_ref):
    deg = jnp.sum(degp_ref[...], axis=1) + 1.0
    dis = lax.rsqrt(deg)
    p1_ref[...] = h1_ref[...] * dis[:, None]
    dis_ref[...] = dis[:, None]


def _tc2_body(aggp_ref, p1_ref, dis_ref, b1_ref, w2_ref, p2_ref):
    a = aggp_ref[0] + aggp_ref[1] + p1_ref[...]
    o1 = a * dis_ref[...] + b1_ref[...]
    r = jnp.maximum(o1, 0.0)
    h2 = jnp.dot(r, w2_ref[...], preferred_element_type=jnp.float32)
    p2_ref[...] = h2 * dis_ref[...]


def _tc3_body(a2p_ref, p2_ref, dis_ref, b2_ref, out_ref):
    ssum = jnp.sum(a2p_ref[...], axis=1)[:, None] + p2_ref[...]
    out_ref[...] = ssum * dis_ref[...] + b2_ref[0, 0]


def kernel(x, edge_index, W1, b1, W2, b2):
    es = edge_index.reshape(2, NW, NCHUNK, CH).transpose(1, 2, 0, 3)

    deg_parts = _sc_deg(es)

    h1 = pl.pallas_call(
        _tc0_body,
        grid=(N // _RB,),
        in_specs=[
            pl.BlockSpec((_RB, D), lambda i: (i, 0)),
            pl.BlockSpec((D, H), lambda i: (0, 0)),
        ],
        out_specs=pl.BlockSpec((_RB, H), lambda i: (i, 0)),
        out_shape=jax.ShapeDtypeStruct((N, H), jnp.float32),
    )(x, W1)

    p1, dis = pl.pallas_call(
        _tc1_body,
        grid=(N // _RB,),
        in_specs=[
            pl.BlockSpec((_RB, NW), lambda i: (i, 0)),
            pl.BlockSpec((_RB, H), lambda i: (i, 0)),
        ],
        out_specs=[
            pl.BlockSpec((_RB, H), lambda i: (i, 0)),
            pl.BlockSpec((_RB, 1), lambda i: (i, 0)),
        ],
        out_shape=[
            jax.ShapeDtypeStruct((N, H), jnp.float32),
            jax.ShapeDtypeStruct((N, 1), jnp.float32),
        ],
    )(deg_parts.T, h1)

    agg_parts = _sc_rows(p1, es)

    p2 = pl.pallas_call(
        _tc2_body,
        grid=(N // _RB,),
        in_specs=[
            pl.BlockSpec((NC, _RB, H), lambda i: (0, i, 0)),
            pl.BlockSpec((_RB, H), lambda i: (i, 0)),
            pl.BlockSpec((_RB, 1), lambda i: (i, 0)),
            pl.BlockSpec((1, H), lambda i: (0, 0)),
            pl.BlockSpec((H, 1), lambda i: (0, 0)),
        ],
        out_specs=pl.BlockSpec((_RB, 1), lambda i: (i, 0)),
        out_shape=jax.ShapeDtypeStruct((N, 1), jnp.float32),
    )(agg_parts, p1, dis, b1.reshape(1, H), W2)

    agg2_parts = _sc_scalar(es, p2.reshape(N))

    out = pl.pallas_call(
        _tc3_body,
        grid=(N // _RB,),
        in_specs=[
            pl.BlockSpec((_RB, NW), lambda i: (i, 0)),
            pl.BlockSpec((_RB, 1), lambda i: (i, 0)),
            pl.BlockSpec((_RB, 1), lambda i: (i, 0)),
            pl.BlockSpec((1, 1), lambda i: (0, 0)),
        ],
        out_specs=pl.BlockSpec((_RB, 1), lambda i: (i, 0)),
        out_shape=jax.ShapeDtypeStruct((N, 1), jnp.float32),
    )(agg2_parts.T, p2, dis, b2.reshape(1, 1))

    return out.reshape(N)


# speedup vs baseline: 1.0077x; 1.0049x over previous
import functools

import jax
import jax.numpy as jnp
from jax import lax
from jax.experimental import pallas as pl
from jax.experimental.pallas import tpu as pltpu
from jax.experimental.pallas import tpu_sc as plsc

N = 10000
E = 320000
D = 128
H = 128

NC = 2
NS = 16
NW = NC * NS
L = 16

EPT = E // NW
CH = 80
NCHUNK = EPT // CH
WPT = 624
TAIL = N - NS * WPT
ZROWS = 16

_mesh = plsc.VectorSubcoreMesh(core_axis_name="c", subcore_axis_name="s")
_sc_params = pltpu.CompilerParams(needs_layout_passes=False)

_Z16 = functools.partial(jnp.zeros, (L,), jnp.float32)


def _zero_1d(ref, nwords):
    def body(i, c):
        ref[pl.ds(i * L, L)] = _Z16()
        return c
    lax.fori_loop(0, nwords // L, body, 0)


@functools.partial(
    pl.kernel,
    out_type=jax.ShapeDtypeStruct((NW, N), jnp.float32),
    mesh=_mesh,
    compiler_params=_sc_params,
    scratch_types=[
        pltpu.VMEM((NCHUNK, 2, CH), jnp.int32),
        pltpu.VMEM((N,), jnp.float32),
    ],
)
def _sc_deg(es_hbm, out_hbm, es_v, acc_v):
    c = lax.axis_index("c")
    s = lax.axis_index("s")
    wid = c * NS + s
    _zero_1d(acc_v, N)
    pltpu.sync_copy(es_hbm.at[wid], es_v)
    ones = jnp.ones((L,), jnp.float32)

    def body(k, carry):
        for j in range(CH // L):
            di = es_v[k, 1, pl.ds(j * L, L)]
            plsc.addupdate_scatter(acc_v, [di], ones)
        return carry

    lax.fori_loop(0, NCHUNK, body, 0)
    pltpu.sync_copy(acc_v, out_hbm.at[wid])


_RING = 3


@functools.partial(
    pl.kernel,
    out_type=jax.ShapeDtypeStruct((NC, N, D), jnp.float32),
    mesh=_mesh,
    compiler_params=_sc_params,
    scratch_types=(
        [pltpu.VMEM_SHARED((N, D), jnp.float32)]
        + [pltpu.VMEM((CH, D), jnp.float32)] * _RING
        + [pltpu.VMEM((2, CH), jnp.int32)] * _RING
        + [pltpu.VMEM((ZROWS, D), jnp.float32)]
        + [pltpu.SemaphoreType.DMA] * (2 * _RING)
    ),
)
def _sc_rows(p1_hbm, es_hbm, out_hbm, acc,
             rows0, rows1, rows2, idx0, idx1, idx2, zbuf,
             g0, g1, g2, i0, i1, i2):
    c = lax.axis_index("c")
    s = lax.axis_index("s")
    wid = c * NS + s
    bufs = ((rows0, idx0, g0, i0), (rows1, idx1, g1, i1), (rows2, idx2, g2, i2))

    def _iload(k, ib, isem):
        pltpu.async_copy(es_hbm.at[wid, k], ib, isem)

    def _iwait(k, ib, isem):
        pltpu.make_async_copy(es_hbm.at[wid, k], ib, isem).wait()

    def _gissue(ib, rb, gsem):
        pltpu.async_copy(p1_hbm.at[ib.at[0]], rb, gsem)

    def _gwait(ib, rb, gsem):
        pltpu.make_async_copy(p1_hbm.at[ib.at[0]], rb, gsem).wait()

    def _scat(ib, rb):
        pltpu.sync_copy(rb, acc.at[ib.at[1]], add=True)

    for b in range(_RING):
        _iload(b, bufs[b][1], bufs[b][3])
    for b in range(2):
        _iwait(b, bufs[b][1], bufs[b][3])
        _gissue(bufs[b][1], bufs[b][0], bufs[b][2])

    def zrow(i, carry):
        for j in range(D // L):
            zbuf[i, pl.ds(j * L, L)] = _Z16()
        return carry

    lax.fori_loop(0, ZROWS, zrow, 0)
    for r in range(WPT // ZROWS):
        pltpu.sync_copy(zbuf, acc.at[pl.ds(s * WPT + r * ZROWS, ZROWS)])

    @pl.when(s == NS - 1)
    def _zero_tail():
        pltpu.sync_copy(zbuf.at[pl.ds(0, TAIL)], acc.at[pl.ds(NS * WPT, TAIL)])

    plsc.subcore_barrier()


    def body(i, carry):
        for b in range(_RING):
            kk = _RING * i + b
            rb, ib, gs, isem = bufs[b]
            rb2, ib2, gs2, isem2 = bufs[(b + 2) % _RING]
            _gwait(ib, rb, gs)
            _scat(ib, rb)

            @pl.when(kk <= NCHUNK - 1 - _RING)
            def _prefetch_idx():
                _iload(kk + _RING, ib, isem)

            @pl.when(kk <= NCHUNK - 3)
            def _prefetch_rows():
                _iwait(kk + 2, ib2, isem2)
                _gissue(ib2, rb2, gs2)
        return carry

    lax.fori_loop(0, NCHUNK // _RING, body, 0)
    for kk in range(_RING * (NCHUNK // _RING), NCHUNK):
        rb, ib, gs, isem = bufs[kk % _RING]
        _gwait(ib, rb, gs)
        _scat(ib, rb)
    plsc.subcore_barrier()
    pltpu.sync_copy(acc.at[pl.ds(s * WPT, WPT)],
                    out_hbm.at[c, pl.ds(s * WPT, WPT)])

    @pl.when(s == NS - 1)
    def _write_tail():
        pltpu.sync_copy(acc.at[pl.ds(NS * WPT, TAIL)],
                        out_hbm.at[c, pl.ds(NS * WPT, TAIL)])


@functools.partial(
    pl.kernel,
    out_type=jax.ShapeDtypeStruct((NW, N), jnp.float32),
    mesh=_mesh,
    compiler_params=_sc_params,
    scratch_types=[
        pltpu.VMEM((NCHUNK, 2, CH), jnp.int32),
        pltpu.VMEM((N,), jnp.float32),
        pltpu.VMEM((N,), jnp.float32),
    ],
)
def _sc_scalar(es_hbm, p2_hbm, out_hbm, es_v, p2_v, acc_v):
    c = lax.axis_index("c")
    s = lax.axis_index("s")
    wid = c * NS + s
    _zero_1d(acc_v, N)
    pltpu.sync_copy(p2_hbm, p2_v)
    pltpu.sync_copy(es_hbm.at[wid], es_v)

    def body(k, carry):
        for j in range(CH // L):
            si = es_v[k, 0, pl.ds(j * L, L)]
            di = es_v[k, 1, pl.ds(j * L, L)]
            vals = plsc.load_gather(p2_v, [si])
            plsc.addupdate_scatter(acc_v, [di], vals)
        return carry

    lax.fori_loop(0, NCHUNK, body, 0)
    pltpu.sync_copy(acc_v, out_hbm.at[wid])


_RB = 2000


def _tc1_body(degp_ref, x_ref, w1_ref, p1_ref, dis_ref):
    deg = jnp.sum(degp_ref[...], axis=1) + 1.0
    dis = lax.rsqrt(deg)
    h = jnp.dot(x_ref[...], w1_ref[...], preferred_element_type=jnp.float32)
    p1_ref[...] = h * dis[:, None]
    dis_ref[...] = dis[:, None]


def _tc2_body(aggp_ref, p1_ref, dis_ref, b1_ref, w2_ref, p2_ref):
    a = aggp_ref[0] + aggp_ref[1] + p1_ref[...]
    o1 = a * dis_ref[...] + b1_ref[...]
    r = jnp.maximum(o1, 0.0)
    h2 = jnp.dot(r, w2_ref[...], preferred_element_type=jnp.float32)
    p2_ref[...] = h2 * dis_ref[...]


def _tc3_body(a2p_ref, p2_ref, dis_ref, b2_ref, out_ref):
    ssum = jnp.sum(a2p_ref[...], axis=1)[:, None] + p2_ref[...]
    out_ref[...] = ssum * dis_ref[...] + b2_ref[0, 0]


def kernel(x, edge_index, W1, b1, W2, b2):
    es = edge_index.reshape(2, NW, NCHUNK, CH).transpose(1, 2, 0, 3)

    deg_parts = _sc_deg(es)

    p1, dis = pl.pallas_call(
        _tc1_body,
        grid=(N // _RB,),
        in_specs=[
            pl.BlockSpec((_RB, NW), lambda i: (i, 0)),
            pl.BlockSpec((_RB, D), lambda i: (i, 0)),
            pl.BlockSpec((D, H), lambda i: (0, 0)),
        ],
        out_specs=[
            pl.BlockSpec((_RB, H), lambda i: (i, 0)),
            pl.BlockSpec((_RB, 1), lambda i: (i, 0)),
        ],
        out_shape=[
            jax.ShapeDtypeStruct((N, H), jnp.float32),
            jax.ShapeDtypeStruct((N, 1), jnp.float32),
        ],
    )(deg_parts.T, x, W1)

    agg_parts = _sc_rows(p1, es)

    p2 = pl.pallas_call(
        _tc2_body,
        grid=(N // _RB,),
        in_specs=[
            pl.BlockSpec((NC, _RB, H), lambda i: (0, i, 0)),
            pl.BlockSpec((_RB, H), lambda i: (i, 0)),
            pl.BlockSpec((_RB, 1), lambda i: (i, 0)),
            pl.BlockSpec((1, H), lambda i: (0, 0)),
            pl.BlockSpec((H, 1), lambda i: (0, 0)),
        ],
        out_specs=pl.BlockSpec((_RB, 1), lambda i: (i, 0)),
        out_shape=jax.ShapeDtypeStruct((N, 1), jnp.float32),
    )(agg_parts, p1, dis, b1.reshape(1, H), W2)

    agg2_parts = _sc_scalar(es, p2.reshape(N))

    out = pl.pallas_call(
        _tc3_body,
        grid=(N // _RB,),
        in_specs=[
            pl.BlockSpec((_RB, NW), lambda i: (i, 0)),
            pl.BlockSpec((_RB, 1), lambda i: (i, 0)),
            pl.BlockSpec((_RB, 1), lambda i: (i, 0)),
            pl.BlockSpec((1, 1), lambda i: (0, 0)),
        ],
        out_specs=pl.BlockSpec((_RB, 1), lambda i: (i, 0)),
        out_shape=jax.ShapeDtypeStruct((N, 1), jnp.float32),
    )(agg2_parts.T, p2, dis, b2.reshape(1, 1))

    return out.reshape(N)
